# baseline (speedup 1.0000x reference)
import jax
import jax.numpy as jnp
from jax import lax
from jax.experimental import pallas as pl
from jax.experimental.pallas import tpu as pltpu
from jax.experimental.pallas import tpu_sc as plsc

_N = 10000
_E = 320000
_D = 128
_K = 88
_NW = 32
_CPT = 114
_EPAD = _NW * _CPT * _K
_NST = 6
_SB = _CPT // _NST
_TRASH = 10232
_RPT = 640
_NPAD = _RPT * 16


def _sc_agg_make(with_counts):
    mesh = plsc.VectorSubcoreMesh(core_axis_name="c", subcore_axis_name="s")
    out_type = jax.ShapeDtypeStruct((2, _NPAD, _D), jnp.float32)
    scratch = [
        pltpu.VMEM((2, _SB, _K), jnp.int32),
        pltpu.VMEM((2, _SB, _K), jnp.int32),
        pltpu.VMEM((2, _K, _D), jnp.float32),
        pltpu.VMEM_SHARED((_NPAD, _D), jnp.float32),
        pltpu.SemaphoreType.DMA,
        pltpu.SemaphoreType.DMA,
        pltpu.SemaphoreType.DMA,
    ]
    if with_counts:
        out_type = [out_type, jax.ShapeDtypeStruct((2, _NPAD), jnp.float32)]
        scratch += [
            pltpu.VMEM((_K,), jnp.float32),
            pltpu.VMEM_SHARED((_NPAD,), jnp.float32),
            pltpu.SemaphoreType.DMA,
        ]

    def body(y, src4, dst4, zrow, zcnt, ones, *rest):
        if with_counts:
            (out, cnt_out, sidx, didx, rows, acc, sem0, sem1, semi,
             ones_v, cacc, semc) = rest
        else:
            out, sidx, didx, rows, acc, sem0, sem1, semi = rest
        c = lax.axis_index("c")
        s = lax.axis_index("s")
        wid = s * 2 + c
        r0 = s * _RPT

        pltpu.sync_copy(zrow, acc.at[pl.ds(r0, _RPT)])
        if with_counts:
            pltpu.sync_copy(zcnt, cacc.at[pl.ds(r0, _RPT)])
            pltpu.sync_copy(ones, ones_v)
        plsc.subcore_barrier()

        def issue_idx(st, b):
            pltpu.async_copy(src4.at[wid, st], sidx.at[b], semi)
            pltpu.async_copy(dst4.at[wid, st], didx.at[b], semi)

        def wait_idx(st, b):
            pltpu.make_async_copy(src4.at[wid, st], sidx.at[b], semi).wait()
            pltpu.make_async_copy(dst4.at[wid, st], didx.at[b], semi).wait()

        def issue(i, ib, buf, sem):
            return pltpu.async_copy(y.at[sidx.at[ib, i]], rows.at[buf], sem)

        def drain(i, ib, buf, sem):
            pltpu.make_async_copy(y.at[sidx.at[ib, i]], rows.at[buf],
                                  sem).wait()

        def scatter(buf, ib, i):
            pltpu.sync_copy(rows.at[buf], acc.at[didx.at[ib, i]], add=True)

        def fire_cnt(ib, i):
            if with_counts:
                pltpu.async_copy(ones_v, cacc.at[didx.at[ib, i]], semc,
                                 add=True)

        def drain_cnt(ib):
            if with_counts:
                pltpu.make_async_copy(ones_v, cacc.at[didx.at[ib, 0]],
                                      semc).wait()

        issue_idx(0, 0)
        wait_idx(0, 0)
        for st in range(_NST):
            ib = st % 2
            if st + 1 < _NST:
                issue_idx(st + 1, 1 - ib)
            issue(0, ib, 0, sem0)

            def pair(t, carry, ib=ib):
                i = 2 * t
                issue(i + 1, ib, 1, sem1)
                fire_cnt(ib, i)
                drain(i, ib, 0, sem0)
                scatter(0, ib, i)

                @pl.when(i + 2 < _SB)
                def _prefetch():
                    issue(i + 2, ib, 0, sem0)

                fire_cnt(ib, i + 1)
                drain(i + 1, ib, 1, sem1)
                scatter(1, ib, i + 1)
                drain_cnt(ib)
                drain_cnt(ib)
                return carry

            lax.fori_loop(0, (_SB - 1) // 2, pair, 0)
            fire_cnt(ib, _SB - 1)
            drain(_SB - 1, ib, 0, sem0)
            scatter(0, ib, _SB - 1)
            drain_cnt(ib)
            if st + 1 < _NST:
                wait_idx(st + 1, 1 - ib)

        plsc.subcore_barrier()
        pltpu.sync_copy(acc.at[pl.ds(r0, _RPT)],
                        out.at[c, pl.ds(r0, _RPT)])
        if with_counts:
            pltpu.sync_copy(cacc.at[pl.ds(r0, _RPT)],
                            cnt_out.at[c, pl.ds(r0, _RPT)])

    return pl.kernel(body, out_type=out_type, mesh=mesh,
                     scratch_types=scratch)


_sc_agg_counts = _sc_agg_make(True)
_sc_agg = _sc_agg_make(False)


def _make_tc_layer(relu):
    n_blocks = 10
    br = 1024

    def body(p_ref, c_ref, x_ref, wl_ref, bl_ref, wr_ref, o_ref):
        cnt = c_ref[0, 0] + c_ref[1, 0]
        recip = 1.0 / jnp.maximum(cnt, 1.0)
        agg = ((p_ref[0] + p_ref[1]) * recip[:, None]).astype(jnp.bfloat16)
        acc = lax.dot_general(agg, wl_ref[...].astype(jnp.bfloat16),
                              (((1,), (1,)), ((), ())),
                              preferred_element_type=jnp.float32)
        acc = acc + bl_ref[...]
        acc = acc + lax.dot_general(x_ref[...].astype(jnp.bfloat16),
                                    wr_ref[...].astype(jnp.bfloat16),
                                    (((1,), (1,)), ((), ())),
                                    preferred_element_type=jnp.float32)
        o_ref[...] = jnp.maximum(acc, 0.0) if relu else acc

    return pl.pallas_call(
        body,
        grid=(n_blocks,),
        in_specs=[
            pl.BlockSpec((2, br, _D), lambda i: (0, i, 0)),
            pl.BlockSpec((2, 1, br), lambda i: (0, 0, i)),
            pl.BlockSpec((br, _D), lambda i: (i, 0)),
            pl.BlockSpec((_D, _D), lambda i: (0, 0)),
            pl.BlockSpec((1, _D), lambda i: (0, 0)),
            pl.BlockSpec((_D, _D), lambda i: (0, 0)),
        ],
        out_specs=pl.BlockSpec((br, _D), lambda i: (i, 0)),
        out_shape=jax.ShapeDtypeStruct((_N, _D), jnp.float32),
    )


_tc_layer_relu = _make_tc_layer(True)
_tc_layer_lin = _make_tc_layer(False)


def kernel(x, edge_index, W1l, b1l, W1r, W2l, b2l, W2r):
    npad_e = _EPAD - _E
    src4 = jnp.concatenate(
        [edge_index[0], jnp.zeros((npad_e,), jnp.int32)]
    ).reshape(_NW, _NST, _SB, _K)
    dst4 = jnp.concatenate(
        [edge_index[1], jnp.full((npad_e,), _TRASH, jnp.int32)]
    ).reshape(_NW, _NST, _SB, _K)
    zrow = jnp.zeros((_RPT, _D), jnp.float32)
    zcnt = jnp.zeros((_RPT,), jnp.float32)
    ones = jnp.ones((_K,), jnp.float32)
    p1, cnts = _sc_agg_counts(x, src4, dst4, zrow, zcnt, ones)
    cnts = cnts.reshape(2, 1, _NPAD)
    h = _tc_layer_relu(p1, cnts, x, W1l, b1l.reshape(1, _D), W1r)
    p2 = _sc_agg(h, src4, dst4, zrow, zcnt, ones)
    logits = _tc_layer_lin(p2, cnts, h, W2l, b2l.reshape(1, _D), W2r)
    return h, logits


# speedup vs baseline: 1.2492x; 1.2492x over previous
import jax
import jax.numpy as jnp
from jax import lax
from jax.experimental import pallas as pl
from jax.experimental.pallas import tpu as pltpu
from jax.experimental.pallas import tpu_sc as plsc

_N = 10000
_E = 320000
_D = 128
_K = 80
_CHUNKS = _E // _K
_NW = 32
_CPT = _CHUNKS // _NW
_NST = 5
_SB = _CPT // _NST
_RPT = 640
_NPAD = _RPT * 16


def _sc_agg_make(with_counts):
    mesh = plsc.VectorSubcoreMesh(core_axis_name="c", subcore_axis_name="s")
    out_type = jax.ShapeDtypeStruct((2, _NPAD, _D), jnp.float32)
    scratch = [
        pltpu.VMEM((2, _SB, _K), jnp.int32),
        pltpu.VMEM((2, _SB, _K), jnp.int32),
        pltpu.VMEM((2, _K, _D), jnp.float32),
        pltpu.VMEM_SHARED((_NPAD, _D), jnp.float32),
        pltpu.SemaphoreType.DMA,
        pltpu.SemaphoreType.DMA,
        pltpu.SemaphoreType.DMA,
    ]
    if with_counts:
        out_type = [out_type, jax.ShapeDtypeStruct((2, _NPAD), jnp.float32)]
        scratch += [
            pltpu.VMEM((_K,), jnp.float32),
            pltpu.VMEM_SHARED((_NPAD,), jnp.float32),
            pltpu.SemaphoreType.DMA,
        ]

    def body(y, src4, dst4, zrow, zcnt, ones, *rest):
        if with_counts:
            (out, cnt_out, sidx, didx, rows, acc, sem0, sem1, semi,
             ones_v, cacc, semc) = rest
        else:
            out, sidx, didx, rows, acc, sem0, sem1, semi = rest
        c = lax.axis_index("c")
        s = lax.axis_index("s")
        wid = s * 2 + c
        r0 = s * _RPT

        pltpu.sync_copy(zrow, acc.at[pl.ds(r0, _RPT)])
        if with_counts:
            pltpu.sync_copy(zcnt, cacc.at[pl.ds(r0, _RPT)])
            pltpu.sync_copy(ones, ones_v)
        plsc.subcore_barrier()

        def issue_idx(st, b):
            pltpu.async_copy(src4.at[wid, st], sidx.at[b], semi)
            pltpu.async_copy(dst4.at[wid, st], didx.at[b], semi)

        def wait_idx(st, b):
            pltpu.make_async_copy(src4.at[wid, st], sidx.at[b], semi).wait()
            pltpu.make_async_copy(dst4.at[wid, st], didx.at[b], semi).wait()

        def issue(i, ib, buf, sem):
            return pltpu.async_copy(y.at[sidx.at[ib, i]], rows.at[buf], sem)

        def drain(i, ib, buf, sem):
            pltpu.make_async_copy(y.at[sidx.at[ib, i]], rows.at[buf],
                                  sem).wait()

        def scatter(buf, ib, i):
            pltpu.sync_copy(rows.at[buf], acc.at[didx.at[ib, i]], add=True)

        def fire_cnt(ib, i):
            if with_counts:
                pltpu.async_copy(ones_v, cacc.at[didx.at[ib, i]], semc,
                                 add=True)

        def drain_cnt(ib):
            if with_counts:
                pltpu.make_async_copy(ones_v, cacc.at[didx.at[ib, 0]],
                                      semc).wait()

        issue_idx(0, 0)
        wait_idx(0, 0)
        for st in range(_NST):
            ib = st % 2
            if st + 1 < _NST:
                issue_idx(st + 1, 1 - ib)
            issue(0, ib, 0, sem0)

            def pair(t, carry, ib=ib):
                i = 2 * t
                issue(i + 1, ib, 1, sem1)
                fire_cnt(ib, i)
                drain(i, ib, 0, sem0)
                scatter(0, ib, i)

                @pl.when(i + 2 < _SB)
                def _prefetch():
                    issue(i + 2, ib, 0, sem0)

                fire_cnt(ib, i + 1)
                drain(i + 1, ib, 1, sem1)
                scatter(1, ib, i + 1)
                drain_cnt(ib)
                drain_cnt(ib)
                return carry

            lax.fori_loop(0, (_SB - 1) // 2, pair, 0)
            fire_cnt(ib, _SB - 1)
            drain(_SB - 1, ib, 0, sem0)
            scatter(0, ib, _SB - 1)
            drain_cnt(ib)
            if st + 1 < _NST:
                wait_idx(st + 1, 1 - ib)

        plsc.subcore_barrier()
        pltpu.sync_copy(acc.at[pl.ds(r0, _RPT)],
                        out.at[c, pl.ds(r0, _RPT)])
        if with_counts:
            pltpu.sync_copy(cacc.at[pl.ds(r0, _RPT)],
                            cnt_out.at[c, pl.ds(r0, _RPT)])

    return pl.kernel(body, out_type=out_type, mesh=mesh,
                     scratch_types=scratch)


_sc_agg_counts = _sc_agg_make(True)
_sc_agg = _sc_agg_make(False)


def _make_tc_layer(relu):
    n_blocks = 10
    br = 1024

    def body(p_ref, c_ref, x_ref, wl_ref, bl_ref, wr_ref, o_ref):
        cnt = c_ref[0, 0] + c_ref[1, 0]
        recip = 1.0 / jnp.maximum(cnt, 1.0)
        agg = ((p_ref[0] + p_ref[1]) * recip[:, None]).astype(jnp.bfloat16)
        acc = lax.dot_general(agg, wl_ref[...].astype(jnp.bfloat16),
                              (((1,), (1,)), ((), ())),
                              preferred_element_type=jnp.float32)
        acc = acc + bl_ref[...]
        acc = acc + lax.dot_general(x_ref[...].astype(jnp.bfloat16),
                                    wr_ref[...].astype(jnp.bfloat16),
                                    (((1,), (1,)), ((), ())),
                                    preferred_element_type=jnp.float32)
        o_ref[...] = jnp.maximum(acc, 0.0) if relu else acc

    return pl.pallas_call(
        body,
        grid=(n_blocks,),
        in_specs=[
            pl.BlockSpec((2, br, _D), lambda i: (0, i, 0)),
            pl.BlockSpec((2, 1, br), lambda i: (0, 0, i)),
            pl.BlockSpec((br, _D), lambda i: (i, 0)),
            pl.BlockSpec((_D, _D), lambda i: (0, 0)),
            pl.BlockSpec((1, _D), lambda i: (0, 0)),
            pl.BlockSpec((_D, _D), lambda i: (0, 0)),
        ],
        out_specs=pl.BlockSpec((br, _D), lambda i: (i, 0)),
        out_shape=jax.ShapeDtypeStruct((_N, _D), jnp.float32),
    )


_tc_layer_relu = _make_tc_layer(True)
_tc_layer_lin = _make_tc_layer(False)


def kernel(x, edge_index, W1l, b1l, W1r, W2l, b2l, W2r):
    src4 = edge_index[0].reshape(_NW, _NST, _SB, _K)
    dst4 = edge_index[1].reshape(_NW, _NST, _SB, _K)
    zrow = jnp.zeros((_RPT, _D), jnp.float32)
    zcnt = jnp.zeros((_RPT,), jnp.float32)
    ones = jnp.ones((_K,), jnp.float32)
    p1, cnts = _sc_agg_counts(x, src4, dst4, zrow, zcnt, ones)
    cnts = cnts.reshape(2, 1, _NPAD)
    h = _tc_layer_relu(p1, cnts, x, W1l, b1l.reshape(1, _D), W1r)
    p2 = _sc_agg(h, src4, dst4, zrow, zcnt, ones)
    logits = _tc_layer_lin(p2, cnts, h, W2l, b2l.reshape(1, _D), W2r)
    return h, logits
